# Initial kernel scaffold; baseline (speedup 1.0000x reference)
import jax
import jax.numpy as jnp
from jax.experimental import pallas as pl


def kernel(x, edge_index, W1, b1, W2, b2):
    raise NotImplementedError("write your pallas kernel here")


# speedup vs baseline: 6.3477x; 6.3477x over previous
import functools

import jax
import jax.numpy as jnp
from jax import lax
from jax.experimental import pallas as pl
from jax.experimental.pallas import tpu as pltpu
from jax.experimental.pallas import tpu_sc as plsc

N_NODES = 10000
D_FEAT = 128
D_DEG = 16

NC = 2
NS = 16
NW = NC * NS

EDGE_BLK = 128
BLKS_PER_TILE = 80
E_PER_TILE = EDGE_BLK * BLKS_PER_TILE
E_PAD = NW * E_PER_TILE

N_ACC = 10240
N_PER_TILE = N_ACC // NS
ROW_CHUNK = 128
N_CHUNKS = N_PER_TILE // ROW_CHUNK


def _make_sc_agg(gather: bool):
    d = D_FEAT
    words = d // 16
    mesh = plsc.VectorSubcoreMesh(
        core_axis_name="c", subcore_axis_name="s",
        num_cores=NC, num_subcores=NS)

    @functools.partial(
        pl.kernel,
        out_type=jax.ShapeDtypeStruct((NC, N_ACC, d), jnp.float32),
        mesh=mesh,
        scratch_types=[
            pltpu.VMEM((EDGE_BLK,), jnp.int32),
            pltpu.VMEM((EDGE_BLK,), jnp.int32),
            pltpu.VMEM((EDGE_BLK, d), jnp.float32),
            pltpu.VMEM((ROW_CHUNK, d), jnp.float32),
            pltpu.VMEM_SHARED((N_ACC, d), jnp.float32),
            pltpu.SemaphoreType.DMA,
        ],
    )
    def agg(*refs):
        if gather:
            g_hbm, src_hbm, dst_hbm, out_hbm = refs[:4]
        else:
            src_hbm, dst_hbm, out_hbm = refs[:3]
            g_hbm = None
        src_v, dst_v, rows_v, buf_v, acc_sh, sem = refs[-6:]
        c = lax.axis_index("c")
        s = lax.axis_index("s")
        wid = c * NS + s
        row0 = s * N_PER_TILE

        if not gather:
            def _ofill(k, _):
                rows_v[k // words, pl.ds((k % words) * 16, 16)] = jnp.ones(
                    (16,), jnp.float32)
                return _
            lax.fori_loop(0, EDGE_BLK * words, _ofill, None)

        @pl.when(c == 0)
        def _():
            def _initg(k, _):
                r = row0 + k * ROW_CHUNK
                if gather:
                    pltpu.sync_copy(g_hbm.at[pl.ds(r, ROW_CHUNK)], buf_v)
                    pltpu.sync_copy(buf_v, acc_sh.at[pl.ds(r, ROW_CHUNK)])
                else:
                    pltpu.sync_copy(rows_v, acc_sh.at[pl.ds(r, ROW_CHUNK)])
                return _
            lax.fori_loop(0, N_CHUNKS, _initg, None)

        @pl.when(c != 0)
        def _():
            def _zfill(k, _):
                buf_v[k // words, pl.ds((k % words) * 16, 16)] = jnp.zeros(
                    (16,), jnp.float32)
                return _
            lax.fori_loop(0, ROW_CHUNK * words, _zfill, None)

            def _initz(k, _):
                pltpu.sync_copy(
                    buf_v, acc_sh.at[pl.ds(row0 + k * ROW_CHUNK, ROW_CHUNK)])
                return _
            lax.fori_loop(0, N_CHUNKS, _initz, None)

        plsc.subcore_barrier()

        e0 = wid * E_PER_TILE

        def _step(i, _):
            off = e0 + i * EDGE_BLK
            pltpu.sync_copy(dst_hbm.at[pl.ds(off, EDGE_BLK)], dst_v)
            if gather:
                pltpu.sync_copy(src_hbm.at[pl.ds(off, EDGE_BLK)], src_v)
                pltpu.async_copy(g_hbm.at[src_v], rows_v, sem).wait()
            pltpu.sync_copy(rows_v, acc_sh.at[dst_v], add=True)
            return _
        lax.fori_loop(0, BLKS_PER_TILE, _step, None)

        plsc.subcore_barrier()

        def _out(k, _):
            r = row0 + k * ROW_CHUNK
            pltpu.sync_copy(acc_sh.at[pl.ds(r, ROW_CHUNK)], buf_v)
            pltpu.sync_copy(buf_v, out_hbm.at[c, pl.ds(r, ROW_CHUNK)])
            return _
        lax.fori_loop(0, N_CHUNKS, _out, None)

    return agg


_sc_agg_feat = _make_sc_agg(True)
_sc_agg_deg = _make_sc_agg(False)


ROW_BLK = 1000
GRID = N_NODES // ROW_BLK


def _dinv(d_ref):
    deg = d_ref[0, :, 0:1] + d_ref[1, :, 0:1]
    return lax.rsqrt(deg)


def _tc_g1_body(x_ref, w_ref, d_ref, o_ref):
    di = _dinv(d_ref)
    o_ref[...] = jnp.dot(x_ref[...], w_ref[...],
                         preferred_element_type=jnp.float32) * di


def _tc_mid_body(p_ref, d_ref, b_ref, w_ref, o_ref):
    di = _dinv(d_ref)
    h = jnp.maximum((p_ref[0] + p_ref[1]) * di + b_ref[...], 0.0)
    o_ref[...] = jnp.dot(h, w_ref[...],
                         preferred_element_type=jnp.float32) * di


def _tc_out_body(p_ref, d_ref, b_ref, o_ref):
    di = _dinv(d_ref)
    o_ref[...] = (p_ref[0] + p_ref[1]) * di + b_ref[...]


_deg_spec = pl.BlockSpec((NC, ROW_BLK, D_FEAT), lambda i: (0, i, 0))
_in_spec = pl.BlockSpec((ROW_BLK, D_FEAT), lambda i: (i, 0))
_row_spec = pl.BlockSpec((ROW_BLK, D_FEAT), lambda i: (i, 0))
_par_spec = pl.BlockSpec((NC, ROW_BLK, D_FEAT), lambda i: (0, i, 0))
_w_spec = pl.BlockSpec((D_FEAT, D_FEAT), lambda i: (0, 0))
_b_spec = pl.BlockSpec((1, D_FEAT), lambda i: (0, 0))

_pad_out = jax.ShapeDtypeStruct((N_ACC, D_FEAT), jnp.float32)

_tc_g1 = pl.pallas_call(
    _tc_g1_body, grid=(GRID,),
    in_specs=[_in_spec, _w_spec, _deg_spec],
    out_specs=_row_spec, out_shape=_pad_out)

_tc_mid = pl.pallas_call(
    _tc_mid_body, grid=(GRID,),
    in_specs=[_par_spec, _deg_spec, _b_spec, _w_spec],
    out_specs=_row_spec, out_shape=_pad_out)

_tc_out = pl.pallas_call(
    _tc_out_body, grid=(GRID,),
    in_specs=[_par_spec, _deg_spec, _b_spec],
    out_specs=_row_spec,
    out_shape=jax.ShapeDtypeStruct((N_NODES, D_FEAT), jnp.float32))


def kernel(x, edge_index, W1, b1, W2, b2):
    src = edge_index[0]
    dst = edge_index[1]
    n_pad = E_PAD - src.shape[0]
    src_p = jnp.concatenate([src, jnp.zeros((n_pad,), jnp.int32)])
    dst_p = jnp.concatenate(
        [dst, jnp.full((n_pad,), N_NODES, jnp.int32)])

    dp = _sc_agg_deg(src_p, dst_p)

    g1 = _tc_g1(x, W1, dp)
    p1 = _sc_agg_feat(g1, src_p, dst_p)
    g2 = _tc_mid(p1, dp, b1.reshape(1, D_FEAT), W2)
    p2 = _sc_agg_feat(g2, src_p, dst_p)
    return _tc_out(p2, dp, b2.reshape(1, D_FEAT))


# speedup vs baseline: 6.7838x; 1.0687x over previous
import functools

import jax
import jax.numpy as jnp
from jax import lax
from jax.experimental import pallas as pl
from jax.experimental.pallas import tpu as pltpu
from jax.experimental.pallas import tpu_sc as plsc

N_NODES = 10000
D_FEAT = 128
D_DEG = 16

NC = 2
NS = 16
NW = NC * NS

EDGE_BLK = 128
BLKS_PER_TILE = 80
E_PER_TILE = EDGE_BLK * BLKS_PER_TILE
E_PAD = NW * E_PER_TILE

N_ACC = 10240
N_PER_TILE = N_ACC // NS
ROW_CHUNK = 128
N_CHUNKS = N_PER_TILE // ROW_CHUNK


def _make_sc_agg(gather: bool):
    d = D_FEAT
    words = d // 16
    mesh = plsc.VectorSubcoreMesh(
        core_axis_name="c", subcore_axis_name="s",
        num_cores=NC, num_subcores=NS)

    @functools.partial(
        pl.kernel,
        out_type=jax.ShapeDtypeStruct((NC, N_ACC, d), jnp.float32),
        mesh=mesh,
        scratch_types=[
            pltpu.VMEM((EDGE_BLK,), jnp.int32),
            pltpu.VMEM((EDGE_BLK,), jnp.int32),
            pltpu.VMEM((EDGE_BLK, d), jnp.float32),
            pltpu.VMEM((ROW_CHUNK, d), jnp.float32),
            pltpu.VMEM_SHARED((N_ACC, d), jnp.float32),
            pltpu.SemaphoreType.DMA,
        ],
    )
    def agg(*refs):
        if gather:
            g_hbm, src_hbm, dst_hbm, out_hbm = refs[:4]
        else:
            src_hbm, dst_hbm, out_hbm = refs[:3]
            g_hbm = None
        src_v, dst_v, rows_v, buf_v, acc_sh, sem = refs[-6:]
        c = lax.axis_index("c")
        s = lax.axis_index("s")
        wid = c * NS + s
        row0 = s * N_PER_TILE

        if not gather:
            def _ofill(k, _):
                rows_v[k // words, pl.ds((k % words) * 16, 16)] = jnp.ones(
                    (16,), jnp.float32)
                return _
            lax.fori_loop(0, EDGE_BLK * words, _ofill, None)

        def _zfill(k, _):
            buf_v[k // words, pl.ds((k % words) * 16, 16)] = jnp.zeros(
                (16,), jnp.float32)
            return _
        lax.fori_loop(0, ROW_CHUNK * words, _zfill, None)

        def _initz(k, _):
            pltpu.sync_copy(
                buf_v, acc_sh.at[pl.ds(row0 + k * ROW_CHUNK, ROW_CHUNK)])
            return _
        lax.fori_loop(0, N_CHUNKS, _initz, None)

        plsc.subcore_barrier()

        e0 = wid * E_PER_TILE

        def _step(i, _):
            off = e0 + i * EDGE_BLK
            pltpu.sync_copy(dst_hbm.at[pl.ds(off, EDGE_BLK)], dst_v)
            if gather:
                pltpu.sync_copy(src_hbm.at[pl.ds(off, EDGE_BLK)], src_v)
                pltpu.async_copy(g_hbm.at[src_v], rows_v, sem).wait()
            pltpu.sync_copy(rows_v, acc_sh.at[dst_v], add=True)
            return _
        lax.fori_loop(0, BLKS_PER_TILE, _step, None)

        plsc.subcore_barrier()

        def _out(k, _):
            r = row0 + k * ROW_CHUNK
            pltpu.sync_copy(acc_sh.at[pl.ds(r, ROW_CHUNK)], buf_v)
            pltpu.sync_copy(buf_v, out_hbm.at[c, pl.ds(r, ROW_CHUNK)])
            return _
        lax.fori_loop(0, N_CHUNKS, _out, None)

    return agg


_sc_agg_feat = _make_sc_agg(True)
_sc_agg_deg = _make_sc_agg(False)


ROW_BLK = 1000
GRID = N_NODES // ROW_BLK


def _dinv(d_ref):
    deg = d_ref[0, :, 0:1] + d_ref[1, :, 0:1] + 1.0
    return lax.rsqrt(deg)


def _tc_g1_body(x_ref, w_ref, d_ref, o_ref):
    di = _dinv(d_ref)
    o_ref[...] = jnp.dot(x_ref[...], w_ref[...],
                         preferred_element_type=jnp.float32) * di


def _tc_mid_body(p_ref, g_ref, d_ref, b_ref, w_ref, o_ref):
    di = _dinv(d_ref)
    h = jnp.maximum(
        (p_ref[0] + p_ref[1] + g_ref[...]) * di + b_ref[...], 0.0)
    o_ref[...] = jnp.dot(h, w_ref[...],
                         preferred_element_type=jnp.float32) * di


def _tc_out_body(p_ref, g_ref, d_ref, b_ref, o_ref):
    di = _dinv(d_ref)
    o_ref[...] = (p_ref[0] + p_ref[1] + g_ref[...]) * di + b_ref[...]


_deg_spec = pl.BlockSpec((NC, ROW_BLK, D_FEAT), lambda i: (0, i, 0))
_in_spec = pl.BlockSpec((ROW_BLK, D_FEAT), lambda i: (i, 0))
_row_spec = pl.BlockSpec((ROW_BLK, D_FEAT), lambda i: (i, 0))
_par_spec = pl.BlockSpec((NC, ROW_BLK, D_FEAT), lambda i: (0, i, 0))
_w_spec = pl.BlockSpec((D_FEAT, D_FEAT), lambda i: (0, 0))
_b_spec = pl.BlockSpec((1, D_FEAT), lambda i: (0, 0))

_pad_out = jax.ShapeDtypeStruct((N_ACC, D_FEAT), jnp.float32)

_tc_g1 = pl.pallas_call(
    _tc_g1_body, grid=(GRID,),
    in_specs=[_in_spec, _w_spec, _deg_spec],
    out_specs=_row_spec, out_shape=_pad_out)

_tc_mid = pl.pallas_call(
    _tc_mid_body, grid=(GRID,),
    in_specs=[_par_spec, _row_spec, _deg_spec, _b_spec, _w_spec],
    out_specs=_row_spec, out_shape=_pad_out)

_tc_out = pl.pallas_call(
    _tc_out_body, grid=(GRID,),
    in_specs=[_par_spec, _row_spec, _deg_spec, _b_spec],
    out_specs=_row_spec,
    out_shape=jax.ShapeDtypeStruct((N_NODES, D_FEAT), jnp.float32))


def kernel(x, edge_index, W1, b1, W2, b2):
    src = edge_index[0]
    dst = edge_index[1]
    n_pad = E_PAD - src.shape[0]
    src_p = jnp.concatenate([src, jnp.zeros((n_pad,), jnp.int32)])
    dst_p = jnp.concatenate(
        [dst, jnp.full((n_pad,), N_NODES, jnp.int32)])

    dp = _sc_agg_deg(src_p, dst_p)

    g1 = _tc_g1(x, W1, dp)
    p1 = _sc_agg_feat(g1, src_p, dst_p)
    g2 = _tc_mid(p1, g1, dp, b1.reshape(1, D_FEAT), W2)
    p2 = _sc_agg_feat(g2, src_p, dst_p)
    return _tc_out(p2, g2, dp, b2.reshape(1, D_FEAT))


# speedup vs baseline: 8.1231x; 1.1974x over previous
import functools

import jax
import jax.numpy as jnp
from jax import lax
from jax.experimental import pallas as pl
from jax.experimental.pallas import tpu as pltpu
from jax.experimental.pallas import tpu_sc as plsc

N_NODES = 10000
D_FEAT = 128
D_DEG = 16

NC = 2
NS = 16
NW = NC * NS

EDGE_BLK = 128
BLKS_PER_TILE = 80
E_PER_TILE = EDGE_BLK * BLKS_PER_TILE
E_PAD = NW * E_PER_TILE

N_ACC = 10240
N_PER_TILE = N_ACC // NS
ROW_CHUNK = 128
N_CHUNKS = N_PER_TILE // ROW_CHUNK


NBLK = BLKS_PER_TILE
DSEM = 4


def _make_sc_agg(gather: bool):
    d = D_FEAT
    words = d // 16
    mesh = plsc.VectorSubcoreMesh(
        core_axis_name="c", subcore_axis_name="s",
        num_cores=NC, num_subcores=NS)

    if gather:
        scratch = [
            pltpu.VMEM((NBLK, EDGE_BLK), jnp.int32),
            pltpu.VMEM((EDGE_BLK, d), jnp.float32),
            pltpu.VMEM((EDGE_BLK, d), jnp.float32),
            pltpu.VMEM((EDGE_BLK,), jnp.int32),
            pltpu.VMEM((EDGE_BLK,), jnp.int32),
            pltpu.VMEM_SHARED((N_ACC, d), jnp.float32),
        ] + [pltpu.SemaphoreType.DMA] * 4
    else:
        scratch = [
            pltpu.VMEM((NBLK, EDGE_BLK), jnp.int32),
            pltpu.VMEM((EDGE_BLK, d), jnp.float32),
            pltpu.VMEM((ROW_CHUNK, d), jnp.float32),
            pltpu.VMEM_SHARED((N_ACC, d), jnp.float32),
        ] + [pltpu.SemaphoreType.DMA] * DSEM

    @functools.partial(
        pl.kernel,
        out_type=jax.ShapeDtypeStruct((NC, N_ACC, d), jnp.float32),
        mesh=mesh,
        scratch_types=scratch,
    )
    def agg(*refs):
        if gather:
            (g_hbm, src3_hbm, dst3_hbm, out_hbm, src2d, rows0, rows1,
             dstb0, dstb1, acc_sh, gsem0, gsem1, dsem0, dsem1) = refs
            rows = [rows0, rows1]
            dstb = [dstb0, dstb1]
            gsem = [gsem0, gsem1]
            dsem = [dsem0, dsem1]
            zbuf = rows0
        else:
            dst3_hbm, out_hbm, dst2d, ones_v, zbuf, acc_sh = refs[:6]
            ssem = list(refs[6:6 + DSEM])
        c = lax.axis_index("c")
        s = lax.axis_index("s")
        wid = c * NS + s
        row0 = s * N_PER_TILE

        if gather:
            pltpu.sync_copy(src3_hbm.at[wid], src2d)
            pltpu.async_copy(dst3_hbm.at[wid, 0], dstb[0], dsem[0])
        else:
            pltpu.sync_copy(dst3_hbm.at[wid], dst2d)

            def _ofill(k, _):
                ones_v[k // words, pl.ds((k % words) * 16, 16)] = jnp.ones(
                    (16,), jnp.float32)
                return _
            lax.fori_loop(0, EDGE_BLK * words, _ofill, None)

        def _zfill(k, _):
            zbuf[k // words, pl.ds((k % words) * 16, 16)] = jnp.zeros(
                (16,), jnp.float32)
            return _
        lax.fori_loop(0, ROW_CHUNK * words, _zfill, None)

        def _initz(k, _):
            pltpu.sync_copy(
                zbuf, acc_sh.at[pl.ds(row0 + k * ROW_CHUNK, ROW_CHUNK)])
            return _
        lax.fori_loop(0, N_CHUNKS, _initz, None)

        if gather:
            pltpu.async_copy(g_hbm.at[src2d.at[0]], rows[0], gsem[0])

        plsc.subcore_barrier()

        if gather:
            def _body(outer, _):
                for b in range(2):
                    i = outer * 2 + b
                    pltpu.make_async_copy(
                        g_hbm.at[src2d.at[i]], rows[b], gsem[b]).wait()

                    @pl.when(i + 1 < NBLK)
                    def _():
                        pltpu.async_copy(
                            g_hbm.at[src2d.at[i + 1]], rows[1 - b],
                            gsem[1 - b])
                        pltpu.async_copy(
                            dst3_hbm.at[wid, i + 1], dstb[1 - b],
                            dsem[1 - b])

                    pltpu.make_async_copy(
                        dst3_hbm.at[wid, i], dstb[b], dsem[b]).wait()
                    pltpu.sync_copy(rows[b], acc_sh.at[dstb[b]], add=True)
                return _
            lax.fori_loop(0, NBLK // 2, _body, None)
        else:
            def _dbody(outer, _):
                for b in range(DSEM):
                    i = outer * DSEM + b

                    @pl.when(i >= DSEM)
                    def _():
                        pltpu.make_async_copy(
                            ones_v, acc_sh.at[dst2d.at[0]], ssem[b]).wait()

                    pltpu.async_copy(
                        ones_v, acc_sh.at[dst2d.at[i]], ssem[b], add=True)
                return _
            lax.fori_loop(0, NBLK // DSEM, _dbody, None)
            for b in range(DSEM):
                pltpu.make_async_copy(
                    ones_v, acc_sh.at[dst2d.at[0]], ssem[b]).wait()

        plsc.subcore_barrier()

        bounce = rows[0] if gather else zbuf

        def _out(k, _):
            r = row0 + k * ROW_CHUNK
            pltpu.sync_copy(acc_sh.at[pl.ds(r, ROW_CHUNK)], bounce)
            pltpu.sync_copy(bounce, out_hbm.at[c, pl.ds(r, ROW_CHUNK)])
            return _
        lax.fori_loop(0, N_CHUNKS, _out, None)

    return agg


_sc_agg_feat = _make_sc_agg(True)
_sc_agg_deg = _make_sc_agg(False)


ROW_BLK = 1000
GRID = N_NODES // ROW_BLK


def _dinv(d_ref):
    deg = d_ref[0, :, 0:1] + d_ref[1, :, 0:1] + 1.0
    return lax.rsqrt(deg)


def _tc_g1_body(x_ref, w_ref, d_ref, o_ref):
    di = _dinv(d_ref)
    o_ref[...] = jnp.dot(x_ref[...], w_ref[...],
                         preferred_element_type=jnp.float32) * di


def _tc_mid_body(p_ref, g_ref, d_ref, b_ref, w_ref, o_ref):
    di = _dinv(d_ref)
    h = jnp.maximum(
        (p_ref[0] + p_ref[1] + g_ref[...]) * di + b_ref[...], 0.0)
    o_ref[...] = jnp.dot(h, w_ref[...],
                         preferred_element_type=jnp.float32) * di


def _tc_out_body(p_ref, g_ref, d_ref, b_ref, o_ref):
    di = _dinv(d_ref)
    o_ref[...] = (p_ref[0] + p_ref[1] + g_ref[...]) * di + b_ref[...]


_deg_spec = pl.BlockSpec((NC, ROW_BLK, D_FEAT), lambda i: (0, i, 0))
_in_spec = pl.BlockSpec((ROW_BLK, D_FEAT), lambda i: (i, 0))
_row_spec = pl.BlockSpec((ROW_BLK, D_FEAT), lambda i: (i, 0))
_par_spec = pl.BlockSpec((NC, ROW_BLK, D_FEAT), lambda i: (0, i, 0))
_w_spec = pl.BlockSpec((D_FEAT, D_FEAT), lambda i: (0, 0))
_b_spec = pl.BlockSpec((1, D_FEAT), lambda i: (0, 0))

_pad_out = jax.ShapeDtypeStruct((N_ACC, D_FEAT), jnp.float32)

_tc_g1 = pl.pallas_call(
    _tc_g1_body, grid=(GRID,),
    in_specs=[_in_spec, _w_spec, _deg_spec],
    out_specs=_row_spec, out_shape=_pad_out)

_tc_mid = pl.pallas_call(
    _tc_mid_body, grid=(GRID,),
    in_specs=[_par_spec, _row_spec, _deg_spec, _b_spec, _w_spec],
    out_specs=_row_spec, out_shape=_pad_out)

_tc_out = pl.pallas_call(
    _tc_out_body, grid=(GRID,),
    in_specs=[_par_spec, _row_spec, _deg_spec, _b_spec],
    out_specs=_row_spec,
    out_shape=jax.ShapeDtypeStruct((N_NODES, D_FEAT), jnp.float32))


def kernel(x, edge_index, W1, b1, W2, b2):
    src = edge_index[0]
    dst = edge_index[1]
    n_pad = E_PAD - src.shape[0]
    src_p = jnp.concatenate(
        [src, jnp.zeros((n_pad,), jnp.int32)]).reshape(NW, NBLK, EDGE_BLK)
    dst_p = jnp.concatenate(
        [dst, jnp.full((n_pad,), N_NODES, jnp.int32)]).reshape(
            NW, NBLK, EDGE_BLK)

    dp = _sc_agg_deg(dst_p)

    g1 = _tc_g1(x, W1, dp)
    p1 = _sc_agg_feat(g1, src_p, dst_p)
    g2 = _tc_mid(p1, g1, dp, b1.reshape(1, D_FEAT), W2)
    p2 = _sc_agg_feat(g2, src_p, dst_p)
    return _tc_out(p2, g2, dp, b2.reshape(1, D_FEAT))


# speedup vs baseline: 10.7152x; 1.3191x over previous
import functools

import jax
import jax.numpy as jnp
from jax import lax
from jax.experimental import pallas as pl
from jax.experimental.pallas import tpu as pltpu
from jax.experimental.pallas import tpu_sc as plsc

N_NODES = 10000
D_FEAT = 128
D_DEG = 16

NC = 2
NS = 16
NW = NC * NS

EDGE_BLK = 128
BLKS_PER_TILE = 80
E_PER_TILE = EDGE_BLK * BLKS_PER_TILE
E_PAD = NW * E_PER_TILE

N_ACC = 10240
N_PER_TILE = N_ACC // NS
ROW_CHUNK = 128
N_CHUNKS = N_PER_TILE // ROW_CHUNK


NBLK = BLKS_PER_TILE
DSEM = 4

NBLK_F = 120
NBLK_S = 2 * NBLK - NBLK_F


def _make_sc_agg(gather: bool):
    d = D_FEAT
    words = d // 16
    mesh = plsc.VectorSubcoreMesh(
        core_axis_name="c", subcore_axis_name="s",
        num_cores=NC, num_subcores=NS)

    if gather:
        scratch = [
            pltpu.VMEM((NBLK_F, EDGE_BLK), jnp.int32),
            pltpu.VMEM((EDGE_BLK, d), jnp.float32),
            pltpu.VMEM((EDGE_BLK, d), jnp.float32),
            pltpu.VMEM((EDGE_BLK,), jnp.int32),
            pltpu.VMEM((EDGE_BLK,), jnp.int32),
            pltpu.VMEM_SHARED((N_ACC, d), jnp.float32),
        ] + [pltpu.SemaphoreType.DMA] * 4
    else:
        scratch = [
            pltpu.VMEM((NBLK, EDGE_BLK), jnp.int32),
            pltpu.VMEM((EDGE_BLK, d), jnp.float32),
            pltpu.VMEM((ROW_CHUNK, d), jnp.float32),
            pltpu.VMEM_SHARED((N_ACC, d), jnp.float32),
        ] + [pltpu.SemaphoreType.DMA] * DSEM

    @functools.partial(
        pl.kernel,
        out_type=jax.ShapeDtypeStruct((NC, N_ACC, d), jnp.float32),
        mesh=mesh,
        scratch_types=scratch,
    )
    def agg(*refs):
        if gather:
            (g_hbm, src3_hbm, dst3_hbm, out_hbm, src2d, rows0, rows1,
             dstb0, dstb1, acc_sh, gsem0, gsem1, dsem0, dsem1) = refs
            rows = [rows0, rows1]
            dstb = [dstb0, dstb1]
            gsem = [gsem0, gsem1]
            dsem = [dsem0, dsem1]
            zbuf = rows0
        else:
            dst3_hbm, out_hbm, dst2d, ones_v, zbuf, acc_sh = refs[:6]
            ssem = list(refs[6:6 + DSEM])
        c = lax.axis_index("c")
        s = lax.axis_index("s")
        wid = c * NS + s
        row0 = s * N_PER_TILE

        if gather:
            nblk = jnp.where(c == 0, NBLK_F, NBLK_S)
            off = c * NBLK_F

            @pl.when(c == 0)
            def _():
                pltpu.sync_copy(src3_hbm.at[s, pl.ds(0, NBLK_F)], src2d)

            @pl.when(c != 0)
            def _():
                pltpu.sync_copy(src3_hbm.at[s, pl.ds(NBLK_F, NBLK_S)],
                                src2d.at[pl.ds(0, NBLK_S)])

            pltpu.async_copy(dst3_hbm.at[s, off], dstb[0], dsem[0])
        else:
            pltpu.sync_copy(dst3_hbm.at[wid], dst2d)

            def _ofill(k, _):
                ones_v[k // words, pl.ds((k % words) * 16, 16)] = jnp.ones(
                    (16,), jnp.float32)
                return _
            lax.fori_loop(0, EDGE_BLK * words, _ofill, None)

        def _zfill(k, _):
            zbuf[k // words, pl.ds((k % words) * 16, 16)] = jnp.zeros(
                (16,), jnp.float32)
            return _
        lax.fori_loop(0, ROW_CHUNK * words, _zfill, None)

        def _initz(k, _):
            pltpu.sync_copy(
                zbuf, acc_sh.at[pl.ds(row0 + k * ROW_CHUNK, ROW_CHUNK)])
            return _
        lax.fori_loop(0, N_CHUNKS, _initz, None)

        if gather:
            pltpu.async_copy(g_hbm.at[src2d.at[0]], rows[0], gsem[0])

        plsc.subcore_barrier()

        if gather:
            def _body(outer, _):
                for b in range(2):
                    i = outer * 2 + b
                    pltpu.make_async_copy(
                        g_hbm.at[src2d.at[i]], rows[b], gsem[b]).wait()

                    @pl.when(i + 1 < nblk)
                    def _():
                        pltpu.async_copy(
                            g_hbm.at[src2d.at[i + 1]], rows[1 - b],
                            gsem[1 - b])
                        pltpu.async_copy(
                            dst3_hbm.at[s, off + i + 1], dstb[1 - b],
                            dsem[1 - b])

                    pltpu.make_async_copy(
                        dst3_hbm.at[s, off + i], dstb[b], dsem[b]).wait()
                    pltpu.sync_copy(rows[b], acc_sh.at[dstb[b]], add=True)
                return _
            lax.fori_loop(0, nblk // 2, _body, None)
        else:
            def _dbody(outer, _):
                for b in range(DSEM):
                    i = outer * DSEM + b

                    @pl.when(i >= DSEM)
                    def _():
                        pltpu.make_async_copy(
                            ones_v, acc_sh.at[dst2d.at[0]], ssem[b]).wait()

                    pltpu.async_copy(
                        ones_v, acc_sh.at[dst2d.at[i]], ssem[b], add=True)
                return _
            lax.fori_loop(0, NBLK // DSEM, _dbody, None)
            for b in range(DSEM):
                pltpu.make_async_copy(
                    ones_v, acc_sh.at[dst2d.at[0]], ssem[b]).wait()

        plsc.subcore_barrier()

        bounce = rows[0] if gather else zbuf

        def _out(k, _):
            r = row0 + k * ROW_CHUNK
            pltpu.sync_copy(acc_sh.at[pl.ds(r, ROW_CHUNK)], bounce)
            pltpu.sync_copy(bounce, out_hbm.at[c, pl.ds(r, ROW_CHUNK)])
            return _
        lax.fori_loop(0, N_CHUNKS, _out, None)

    return agg


_sc_agg_feat = _make_sc_agg(True)
_sc_agg_deg = _make_sc_agg(False)


ROW_BLK = 1000
GRID = N_NODES // ROW_BLK


def _dinv(d_ref):
    deg = d_ref[0, :, 0:1] + d_ref[1, :, 0:1] + 1.0
    return lax.rsqrt(deg)


def _tc_g1_body(x_ref, w_ref, d_ref, o_ref):
    di = _dinv(d_ref)
    o_ref[...] = jnp.dot(x_ref[...], w_ref[...],
                         preferred_element_type=jnp.float32) * di


def _tc_mid_body(p_ref, g_ref, d_ref, b_ref, w_ref, o_ref):
    di = _dinv(d_ref)
    h = jnp.maximum(
        (p_ref[0] + p_ref[1] + g_ref[...]) * di + b_ref[...], 0.0)
    o_ref[...] = jnp.dot(h, w_ref[...],
                         preferred_element_type=jnp.float32) * di


def _tc_out_body(p_ref, g_ref, d_ref, b_ref, o_ref):
    di = _dinv(d_ref)
    o_ref[...] = (p_ref[0] + p_ref[1] + g_ref[...]) * di + b_ref[...]


_deg_spec = pl.BlockSpec((NC, ROW_BLK, D_FEAT), lambda i: (0, i, 0))
_in_spec = pl.BlockSpec((ROW_BLK, D_FEAT), lambda i: (i, 0))
_row_spec = pl.BlockSpec((ROW_BLK, D_FEAT), lambda i: (i, 0))
_par_spec = pl.BlockSpec((NC, ROW_BLK, D_FEAT), lambda i: (0, i, 0))
_w_spec = pl.BlockSpec((D_FEAT, D_FEAT), lambda i: (0, 0))
_b_spec = pl.BlockSpec((1, D_FEAT), lambda i: (0, 0))

_pad_out = jax.ShapeDtypeStruct((N_ACC, D_FEAT), jnp.float32)

_tc_g1 = pl.pallas_call(
    _tc_g1_body, grid=(GRID,),
    in_specs=[_in_spec, _w_spec, _deg_spec],
    out_specs=_row_spec, out_shape=_pad_out)

_tc_mid = pl.pallas_call(
    _tc_mid_body, grid=(GRID,),
    in_specs=[_par_spec, _row_spec, _deg_spec, _b_spec, _w_spec],
    out_specs=_row_spec, out_shape=_pad_out)

_tc_out = pl.pallas_call(
    _tc_out_body, grid=(GRID,),
    in_specs=[_par_spec, _row_spec, _deg_spec, _b_spec],
    out_specs=_row_spec,
    out_shape=jax.ShapeDtypeStruct((N_NODES, D_FEAT), jnp.float32))


def kernel(x, edge_index, W1, b1, W2, b2):
    src = edge_index[0]
    dst = edge_index[1]
    n_pad = E_PAD - src.shape[0]
    src_p = jnp.concatenate([src, jnp.zeros((n_pad,), jnp.int32)])
    dst_p = jnp.concatenate([dst, jnp.full((n_pad,), N_NODES, jnp.int32)])
    dst_deg = dst_p.reshape(NW, NBLK, EDGE_BLK)
    src_f = src_p.reshape(NS, 2 * NBLK, EDGE_BLK)
    dst_f = dst_p.reshape(NS, 2 * NBLK, EDGE_BLK)

    dp = _sc_agg_deg(dst_deg)

    g1 = _tc_g1(x, W1, dp)
    p1 = _sc_agg_feat(g1, src_f, dst_f)
    g2 = _tc_mid(p1, g1, dp, b1.reshape(1, D_FEAT), W2)
    p2 = _sc_agg_feat(g2, src_f, dst_f)
    return _tc_out(p2, g2, dp, b2.reshape(1, D_FEAT))


# speedup vs baseline: 11.1335x; 1.0390x over previous
import functools

import jax
import jax.numpy as jnp
from jax import lax
from jax.experimental import pallas as pl
from jax.experimental.pallas import tpu as pltpu
from jax.experimental.pallas import tpu_sc as plsc

N_NODES = 10000
D_FEAT = 128
D_DEG = 16

NC = 2
NS = 16
NW = NC * NS

EDGE_BLK = 128
BLKS_PER_TILE = 80
E_PER_TILE = EDGE_BLK * BLKS_PER_TILE
E_PAD = NW * E_PER_TILE

N_ACC = 10240
N_PER_TILE = N_ACC // NS
ROW_CHUNK = 128
N_CHUNKS = N_PER_TILE // ROW_CHUNK


NBLK = BLKS_PER_TILE
DSEM = 4

FB = 64
FBLK_TOT = E_PAD // (NS * FB)
NBLK_F = 240
NBLK_S = FBLK_TOT - NBLK_F
FNBUF = 4
FLOOK = 2
F_CHUNK = 64
F_NCH = N_PER_TILE // F_CHUNK


def _make_sc_agg(gather: bool):
    d = D_FEAT
    words = d // 16
    mesh = plsc.VectorSubcoreMesh(
        core_axis_name="c", subcore_axis_name="s",
        num_cores=NC, num_subcores=NS)

    if gather:
        scratch = (
            [pltpu.VMEM((FB, d), jnp.float32)] * FNBUF
            + [pltpu.VMEM((FB,), jnp.int32)] * FNBUF
            + [pltpu.VMEM((FB,), jnp.int32)] * FNBUF
            + [pltpu.VMEM_SHARED((N_ACC, d), jnp.float32)]
            + [pltpu.SemaphoreType.DMA] * (4 * FNBUF))
    else:
        scratch = [
            pltpu.VMEM((NBLK, EDGE_BLK), jnp.int32),
            pltpu.VMEM((EDGE_BLK, d), jnp.float32),
            pltpu.VMEM((ROW_CHUNK, d), jnp.float32),
            pltpu.VMEM_SHARED((N_ACC, d), jnp.float32),
        ] + [pltpu.SemaphoreType.DMA] * DSEM

    @functools.partial(
        pl.kernel,
        out_type=jax.ShapeDtypeStruct((NC, N_ACC, d), jnp.float32),
        mesh=mesh,
        scratch_types=scratch,
    )
    def agg(*refs):
        if gather:
            g_hbm, src3_hbm, dst3_hbm, out_hbm = refs[:4]
            rows = list(refs[4:4 + FNBUF])
            dstb = list(refs[4 + FNBUF:4 + 2 * FNBUF])
            srcb = list(refs[4 + 2 * FNBUF:4 + 3 * FNBUF])
            acc_sh = refs[4 + 3 * FNBUF]
            sems = list(refs[5 + 3 * FNBUF:])
            gsem = sems[:FNBUF]
            dsem = sems[FNBUF:2 * FNBUF]
            rsem = sems[2 * FNBUF:3 * FNBUF]
            ssem = sems[3 * FNBUF:4 * FNBUF]
        else:
            dst3_hbm, out_hbm, dst2d, ones_v, zbuf, acc_sh = refs[:6]
            ssem = list(refs[6:6 + DSEM])
        c = lax.axis_index("c")
        s = lax.axis_index("s")
        wid = c * NS + s
        row0 = s * N_PER_TILE

        if gather:
            nblk = jnp.where(c == 0, NBLK_F, NBLK_S)
            off = c * NBLK_F
            chunk, nch = F_CHUNK, F_NCH
            zbuf = rows[0]
        else:
            chunk, nch = ROW_CHUNK, N_CHUNKS
            pltpu.sync_copy(dst3_hbm.at[wid], dst2d)

            def _ofill(k, _):
                ones_v[k // words, pl.ds((k % words) * 16, 16)] = jnp.ones(
                    (16,), jnp.float32)
                return _
            lax.fori_loop(0, EDGE_BLK * words, _ofill, None)

        def _zfill(k, _):
            zbuf[k // words, pl.ds((k % words) * 16, 16)] = jnp.zeros(
                (16,), jnp.float32)
            return _
        lax.fori_loop(0, chunk * words, _zfill, None)

        def _initz(k, _):
            pltpu.sync_copy(zbuf, acc_sh.at[pl.ds(row0 + k * chunk, chunk)])
            return _
        lax.fori_loop(0, nch, _initz, None)

        if gather:
            for b in range(FNBUF):
                pltpu.async_copy(src3_hbm.at[s, off + b], srcb[b], rsem[b])
            for b in range(FLOOK):
                pltpu.make_async_copy(
                    src3_hbm.at[s, off + b], srcb[b], rsem[b]).wait()
                pltpu.async_copy(g_hbm.at[srcb[b]], rows[b], gsem[b])
                pltpu.async_copy(dst3_hbm.at[s, off + b], dstb[b], dsem[b])

        plsc.subcore_barrier()

        if gather:
            def _body(outer, _):
                for b in range(FNBUF):
                    i = outer * FNBUF + b
                    bj = (b + FLOOK) % FNBUF
                    j = i + FLOOK
                    pltpu.make_async_copy(
                        g_hbm.at[srcb[b]], rows[b], gsem[b]).wait()
                    pltpu.make_async_copy(
                        dst3_hbm.at[s, off + i], dstb[b], dsem[b]).wait()
                    pltpu.async_copy(
                        rows[b], acc_sh.at[dstb[b]], ssem[b], add=True)

                    @pl.when(i + FNBUF < nblk)
                    def _():
                        pltpu.async_copy(
                            src3_hbm.at[s, off + i + FNBUF], srcb[b],
                            rsem[b])

                    @pl.when(jnp.logical_and(j >= FNBUF, j < nblk))
                    def _():
                        pltpu.make_async_copy(
                            rows[bj], acc_sh.at[dstb[bj]], ssem[bj]).wait()

                    @pl.when(j < nblk)
                    def _():
                        pltpu.make_async_copy(
                            src3_hbm.at[s, off + j], srcb[bj],
                            rsem[bj]).wait()
                        pltpu.async_copy(
                            g_hbm.at[srcb[bj]], rows[bj], gsem[bj])
                        pltpu.async_copy(
                            dst3_hbm.at[s, off + j], dstb[bj], dsem[bj])
                return _
            lax.fori_loop(0, nblk // FNBUF, _body, None)
            for b in range(FNBUF):
                pltpu.make_async_copy(
                    rows[b], acc_sh.at[dstb[b]], ssem[b]).wait()
        else:
            def _dbody(outer, _):
                for b in range(DSEM):
                    i = outer * DSEM + b

                    @pl.when(i >= DSEM)
                    def _():
                        pltpu.make_async_copy(
                            ones_v, acc_sh.at[dst2d.at[0]], ssem[b]).wait()

                    pltpu.async_copy(
                        ones_v, acc_sh.at[dst2d.at[i]], ssem[b], add=True)
                return _
            lax.fori_loop(0, NBLK // DSEM, _dbody, None)
            for b in range(DSEM):
                pltpu.make_async_copy(
                    ones_v, acc_sh.at[dst2d.at[0]], ssem[b]).wait()

        plsc.subcore_barrier()

        bounce = rows[0] if gather else zbuf

        def _out(k, _):
            r = row0 + k * chunk
            pltpu.sync_copy(acc_sh.at[pl.ds(r, chunk)], bounce)
            pltpu.sync_copy(bounce, out_hbm.at[c, pl.ds(r, chunk)])
            return _
        lax.fori_loop(0, nch, _out, None)

    return agg


_sc_agg_feat = _make_sc_agg(True)
_sc_agg_deg = _make_sc_agg(False)


ROW_BLK = 1000
GRID = N_NODES // ROW_BLK


def _dinv(d_ref):
    deg = d_ref[0, :, 0:1] + d_ref[1, :, 0:1] + 1.0
    return lax.rsqrt(deg)


def _tc_g1_body(x_ref, w_ref, d_ref, o_ref):
    di = _dinv(d_ref)
    o_ref[...] = jnp.dot(x_ref[...], w_ref[...],
                         preferred_element_type=jnp.float32) * di


def _tc_mid_body(p_ref, g_ref, d_ref, b_ref, w_ref, o_ref):
    di = _dinv(d_ref)
    h = jnp.maximum(
        (p_ref[0] + p_ref[1] + g_ref[...]) * di + b_ref[...], 0.0)
    o_ref[...] = jnp.dot(h, w_ref[...],
                         preferred_element_type=jnp.float32) * di


def _tc_out_body(p_ref, g_ref, d_ref, b_ref, o_ref):
    di = _dinv(d_ref)
    o_ref[...] = (p_ref[0] + p_ref[1] + g_ref[...]) * di + b_ref[...]


_deg_spec = pl.BlockSpec((NC, ROW_BLK, D_FEAT), lambda i: (0, i, 0))
_in_spec = pl.BlockSpec((ROW_BLK, D_FEAT), lambda i: (i, 0))
_row_spec = pl.BlockSpec((ROW_BLK, D_FEAT), lambda i: (i, 0))
_par_spec = pl.BlockSpec((NC, ROW_BLK, D_FEAT), lambda i: (0, i, 0))
_w_spec = pl.BlockSpec((D_FEAT, D_FEAT), lambda i: (0, 0))
_b_spec = pl.BlockSpec((1, D_FEAT), lambda i: (0, 0))

_pad_out = jax.ShapeDtypeStruct((N_ACC, D_FEAT), jnp.float32)

_tc_g1 = pl.pallas_call(
    _tc_g1_body, grid=(GRID,),
    in_specs=[_in_spec, _w_spec, _deg_spec],
    out_specs=_row_spec, out_shape=_pad_out)

_tc_mid = pl.pallas_call(
    _tc_mid_body, grid=(GRID,),
    in_specs=[_par_spec, _row_spec, _deg_spec, _b_spec, _w_spec],
    out_specs=_row_spec, out_shape=_pad_out)

_tc_out = pl.pallas_call(
    _tc_out_body, grid=(GRID,),
    in_specs=[_par_spec, _row_spec, _deg_spec, _b_spec],
    out_specs=_row_spec,
    out_shape=jax.ShapeDtypeStruct((N_NODES, D_FEAT), jnp.float32))


def kernel(x, edge_index, W1, b1, W2, b2):
    src = edge_index[0]
    dst = edge_index[1]
    n_pad = E_PAD - src.shape[0]
    src_p = jnp.concatenate([src, jnp.zeros((n_pad,), jnp.int32)])
    dst_p = jnp.concatenate([dst, jnp.full((n_pad,), N_NODES, jnp.int32)])
    dst_deg = dst_p.reshape(NW, NBLK, EDGE_BLK)
    src_f = src_p.reshape(NS, FBLK_TOT, FB)
    dst_f = dst_p.reshape(NS, FBLK_TOT, FB)

    dp = _sc_agg_deg(dst_deg)

    g1 = _tc_g1(x, W1, dp)
    p1 = _sc_agg_feat(g1, src_f, dst_f)
    g2 = _tc_mid(p1, g1, dp, b1.reshape(1, D_FEAT), W2)
    p2 = _sc_agg_feat(g2, src_f, dst_f)
    return _tc_out(p2, g2, dp, b2.reshape(1, D_FEAT))
